# baseline (speedup 1.0000x reference)
import functools

import jax
import jax.numpy as jnp
from jax import lax
from jax.experimental import pallas as pl
from jax.experimental.pallas import tpu as pltpu
from jax.experimental.pallas import tpu_sc as plsc

SEQ = 8192
BATCH = 4
D = 768
HD = D // 2
L = 16

NC = 2
NS = 16
NW = NC * NS
RPW = SEQ // NW
CH = 16
NCH = RPW // CH


def _sc_body(x_hbm, posx_hbm, posy_hbm, pe1_hbm, pe2_hbm, out_hbm,
             rawx_v, rawy_v, idx1_v, idx2_v, xbuf, p1buf, p2buf, sem):
    wid = lax.axis_index("s") * NC + lax.axis_index("c")
    base = wid * RPW

    pltpu.sync_copy(posx_hbm.at[pl.ds(base, RPW)], rawx_v)
    pltpu.sync_copy(posy_hbm.at[pl.ds(base, RPW)], rawy_v)
    for t in range(NCH):
        vx = rawx_v[pl.ds(t * L, L)]
        vy = rawy_v[pl.ds(t * L, L)]
        idx1_v[t, :] = vx * 2
        idx2_v[t, :] = vy * 2 + 1

    def chunk(c, _):
        rs = (base + c * CH) * BATCH
        pltpu.sync_copy(x_hbm.at[pl.ds(rs, CH * BATCH)], xbuf)
        pltpu.async_copy(pe1_hbm.at[idx1_v.at[c]], p1buf, sem).wait()
        pltpu.async_copy(pe2_hbm.at[idx2_v.at[c]], p2buf, sem).wait()
        def row_add(r, _):
            row0 = r * BATCH
            for j in range(HD // L):
                v1 = p1buf[r, pl.ds(j * L, L)]
                v2 = p2buf[r, pl.ds(j * L, L)]
                for b in range(BATCH):
                    plsc.addupdate(xbuf.at[row0 + b, pl.ds(j * L, L)], v1)
                    plsc.addupdate(xbuf.at[row0 + b, pl.ds(HD + j * L, L)], v2)
            return 0

        lax.fori_loop(0, CH, row_add, 0)
        pltpu.sync_copy(xbuf, out_hbm.at[pl.ds(rs, CH * BATCH)])
        return 0

    lax.fori_loop(0, NCH, chunk, 0)


@jax.jit
def kernel(x, pos_x, pos_y, pe1, pe2):
    x2 = x.reshape(SEQ * BATCH, D)
    pe1h = pe1.reshape(2 * pe1.shape[0], HD)
    pe2h = pe2.reshape(2 * pe2.shape[0], HD)

    run = pl.kernel(
        _sc_body,
        out_type=jax.ShapeDtypeStruct((SEQ * BATCH, D), jnp.float32),
        mesh=plsc.VectorSubcoreMesh(core_axis_name="c", subcore_axis_name="s"),
        scratch_types=[
            pltpu.VMEM((RPW,), jnp.int32),
            pltpu.VMEM((RPW,), jnp.int32),
            pltpu.VMEM((NCH, CH), jnp.int32),
            pltpu.VMEM((NCH, CH), jnp.int32),
            pltpu.VMEM((CH * BATCH, D), jnp.float32),
            pltpu.VMEM((CH, HD), jnp.float32),
            pltpu.VMEM((CH, HD), jnp.float32),
            pltpu.SemaphoreType.DMA,
        ],
    )
    out2 = run(x2, pos_x, pos_y, pe1h, pe2h)
    return out2.reshape(SEQ, BATCH, D)


# speedup vs baseline: 18.4645x; 18.4645x over previous
import jax
import jax.numpy as jnp
from jax import lax
from jax.experimental import pallas as pl
from jax.experimental.pallas import tpu as pltpu
from jax.experimental.pallas import tpu_sc as plsc

SEQ = 8192
BATCH = 4
D = 768
HD = D // 2
L = 16

NC = 2
NS = 16
NW = NC * NS
RPW = SEQ // NW
CH = 16
NCH = RPW // CH


def _sc_body(x_hbm, posx_hbm, posy_hbm, pe1_hbm, pe2_hbm, out_hbm,
             rawx_v, rawy_v, idx1_v, idx2_v, xbuf, p1buf, p2buf, sem):
    wid = lax.axis_index("s") * NC + lax.axis_index("c")
    base = wid * RPW

    pltpu.sync_copy(posx_hbm.at[pl.ds(base, RPW)], rawx_v)
    pltpu.sync_copy(posy_hbm.at[pl.ds(base, RPW)], rawy_v)
    for t in range(NCH):
        idx1_v[t, :] = rawx_v[pl.ds(t * L, L)]
        idx2_v[t, :] = rawy_v[pl.ds(t * L, L)]

    def chunk(c, _):
        r0 = base + c * CH
        cx = pltpu.async_copy(x_hbm.at[pl.ds(r0, CH)], xbuf, sem)
        c1 = pltpu.async_copy(pe1_hbm.at[idx1_v.at[c]], p1buf, sem)
        c2 = pltpu.async_copy(pe2_hbm.at[idx2_v.at[c]], p2buf, sem)
        cx.wait()
        c1.wait()
        c2.wait()

        def row_add(r, _):
            for j in range(HD // L):
                v1 = p1buf[r, 0, pl.ds(j * L, L)]
                v2 = p2buf[r, 0, pl.ds(HD + j * L, L)]
                for b in range(BATCH):
                    plsc.addupdate(xbuf.at[r, b, pl.ds(j * L, L)], v1)
                    plsc.addupdate(xbuf.at[r, b, pl.ds(HD + j * L, L)], v2)
            return 0

        lax.fori_loop(0, CH, row_add, 0)
        pltpu.sync_copy(xbuf, out_hbm.at[pl.ds(r0, CH)])
        return 0

    lax.fori_loop(0, NCH, chunk, 0)


@jax.jit
def kernel(x, pos_x, pos_y, pe1, pe2):
    run = pl.kernel(
        _sc_body,
        out_type=jax.ShapeDtypeStruct((SEQ, BATCH, D), jnp.float32),
        mesh=plsc.VectorSubcoreMesh(core_axis_name="c", subcore_axis_name="s"),
        scratch_types=[
            pltpu.VMEM((RPW,), jnp.int32),
            pltpu.VMEM((RPW,), jnp.int32),
            pltpu.VMEM((NCH, CH), jnp.int32),
            pltpu.VMEM((NCH, CH), jnp.int32),
            pltpu.VMEM((CH, BATCH, D), jnp.float32),
            pltpu.VMEM((CH, 1, D), jnp.float32),
            pltpu.VMEM((CH, 1, D), jnp.float32),
            pltpu.SemaphoreType.DMA,
        ],
    )
    return run(x, pos_x, pos_y, pe1, pe2)


# speedup vs baseline: 23.5707x; 1.2765x over previous
import jax
import jax.numpy as jnp
from jax import lax
from jax.experimental import pallas as pl
from jax.experimental.pallas import tpu as pltpu
from jax.experimental.pallas import tpu_sc as plsc

SEQ = 8192
BATCH = 4
D = 768
HD = D // 2
L = 16

NC = 2
NS = 16
NW = NC * NS
RPW = SEQ // NW
CH = 8
NCH = RPW // CH
NIT = NCH // 2


def _sc_body(x_hbm, posx_hbm, posy_hbm, pe1_hbm, pe2_hbm, out_hbm,
             rawx_v, rawy_v,
             xb0, xb1, p10, p11, p20, p21,
             isem0, isem1, osem0, osem1):
    wid = lax.axis_index("s") * NC + lax.axis_index("c")
    base = wid * RPW

    pltpu.sync_copy(posx_hbm.at[pl.ds(base, RPW)], rawx_v)
    pltpu.sync_copy(posy_hbm.at[pl.ds(base, RPW)], rawy_v)

    def issue_in(c, xb, p1, p2, isem):
        r0 = base + c * CH
        pltpu.async_copy(x_hbm.at[pl.ds(r0, CH)], xb, isem)
        pltpu.async_copy(pe1_hbm.at[rawx_v.at[pl.ds(c * CH, CH)]], p1, isem)
        pltpu.async_copy(pe2_hbm.at[rawy_v.at[pl.ds(c * CH, CH)]], p2, isem)

    def wait_in(c, xb, p1, p2, isem):
        r0 = base + c * CH
        pltpu.make_async_copy(x_hbm.at[pl.ds(r0, CH)], xb, isem).wait()
        pltpu.make_async_copy(pe1_hbm.at[rawx_v.at[pl.ds(c * CH, CH)]], p1, isem).wait()
        pltpu.make_async_copy(pe2_hbm.at[rawy_v.at[pl.ds(c * CH, CH)]], p2, isem).wait()

    def add_pe(xb, p1, p2):
        def row_add(r, _):
            for j in range(HD // L):
                v1 = p1[r, 0, pl.ds(j * L, L)]
                v2 = p2[r, 0, pl.ds(HD + j * L, L)]
                for b in range(BATCH):
                    plsc.addupdate(xb.at[r, b, pl.ds(j * L, L)], v1)
                    plsc.addupdate(xb.at[r, b, pl.ds(HD + j * L, L)], v2)
            return 0

        lax.fori_loop(0, CH, row_add, 0)

    def issue_out(c, xb, osem):
        r0 = base + c * CH
        pltpu.async_copy(xb, out_hbm.at[pl.ds(r0, CH)], osem)

    def wait_out(c, xb, osem):
        r0 = base + c * CH
        pltpu.make_async_copy(xb, out_hbm.at[pl.ds(r0, CH)], osem).wait()

    issue_in(0, xb0, p10, p20, isem0)
    issue_in(1, xb1, p11, p21, isem1)

    def body(it, _):
        g0 = 2 * it
        g1 = g0 + 1
        wait_in(g0, xb0, p10, p20, isem0)
        add_pe(xb0, p10, p20)
        issue_out(g0, xb0, osem0)

        wait_in(g1, xb1, p11, p21, isem1)
        add_pe(xb1, p11, p21)
        issue_out(g1, xb1, osem1)

        wait_out(g0, xb0, osem0)

        @pl.when(it < NIT - 1)
        def _():
            issue_in(g0 + 2, xb0, p10, p20, isem0)

        wait_out(g1, xb1, osem1)

        @pl.when(it < NIT - 1)
        def _():
            issue_in(g1 + 2, xb1, p11, p21, isem1)

        return 0

    lax.fori_loop(0, NIT, body, 0)


@jax.jit
def kernel(x, pos_x, pos_y, pe1, pe2):
    run = pl.kernel(
        _sc_body,
        out_type=jax.ShapeDtypeStruct((SEQ, BATCH, D), jnp.float32),
        mesh=plsc.VectorSubcoreMesh(core_axis_name="c", subcore_axis_name="s"),
        scratch_types=[
            pltpu.VMEM((RPW,), jnp.int32),
            pltpu.VMEM((RPW,), jnp.int32),
            pltpu.VMEM((CH, BATCH, D), jnp.float32),
            pltpu.VMEM((CH, BATCH, D), jnp.float32),
            pltpu.VMEM((CH, 1, D), jnp.float32),
            pltpu.VMEM((CH, 1, D), jnp.float32),
            pltpu.VMEM((CH, 1, D), jnp.float32),
            pltpu.VMEM((CH, 1, D), jnp.float32),
            pltpu.SemaphoreType.DMA,
            pltpu.SemaphoreType.DMA,
            pltpu.SemaphoreType.DMA,
            pltpu.SemaphoreType.DMA,
        ],
    )
    return run(x, pos_x, pos_y, pe1, pe2)


# speedup vs baseline: 26.5925x; 1.1282x over previous
import jax
import jax.numpy as jnp
from jax import lax
from jax.experimental import pallas as pl
from jax.experimental.pallas import tpu as pltpu
from jax.experimental.pallas import tpu_sc as plsc

SEQ = 8192
BATCH = 4
D = 768
HD = D // 2
L = 16

NC = 2
NS = 16
NW = NC * NS
RPW = SEQ // NW
CH = 8
NCH = RPW // CH
NIT = NCH // 4


def _sc_body(x_hbm, posx_hbm, posy_hbm, pe1_hbm, pe2_hbm, out_hbm,
             rawx_v, rawy_v,
             xb0, xb1, xb2, xb3, p10, p11, p20, p21,
             ix0, ix1, ix2, ix3, ip0, ip1, os0, os1, os2, os3):
    wid = lax.axis_index("s") * NC + lax.axis_index("c")
    base = wid * RPW

    xbs = (xb0, xb1, xb2, xb3)
    ixs = (ix0, ix1, ix2, ix3)
    oss = (os0, os1, os2, os3)
    p1s = (p10, p11)
    p2s = (p20, p21)
    ips = (ip0, ip1)

    pltpu.sync_copy(posx_hbm.at[pl.ds(base, RPW)], rawx_v)
    pltpu.sync_copy(posy_hbm.at[pl.ds(base, RPW)], rawy_v)

    def x_src(c):
        return x_hbm.at[pl.ds(base + c * CH, CH)]

    def out_dst(c):
        return out_hbm.at[pl.ds(base + c * CH, CH)]

    def pe_srcs(c):
        return (pe1_hbm.at[rawx_v.at[pl.ds(c * CH, CH)]],
                pe2_hbm.at[rawy_v.at[pl.ds(c * CH, CH)]])

    def issue_pe(c, k2):
        s1, s2 = pe_srcs(c)
        pltpu.async_copy(s1, p1s[k2], ips[k2])
        pltpu.async_copy(s2, p2s[k2], ips[k2])

    def add_pe(xb, p1, p2):
        def row_add(r, _):
            for j in range(HD // L):
                v1 = p1[r, 0, pl.ds(j * L, L)]
                v2 = p2[r, 0, pl.ds(HD + j * L, L)]
                for b in range(BATCH):
                    plsc.addupdate(xb.at[r, b, pl.ds(j * L, L)], v1)
                    plsc.addupdate(xb.at[r, b, pl.ds(HD + j * L, L)], v2)
            return 0

        lax.fori_loop(0, CH, row_add, 0)

    pltpu.async_copy(x_src(0), xb0, ix0)
    pltpu.async_copy(x_src(1), xb1, ix1)
    issue_pe(0, 0)
    issue_pe(1, 1)

    def body(it, _):
        g0 = 4 * it
        for k in range(4):
            g = g0 + k
            k2 = k % 2
            kx2 = (k + 2) % 4
            pltpu.make_async_copy(x_src(g), xbs[k], ixs[k]).wait()
            s1, s2 = pe_srcs(g)
            pltpu.make_async_copy(s1, p1s[k2], ips[k2]).wait()
            pltpu.make_async_copy(s2, p2s[k2], ips[k2]).wait()
            add_pe(xbs[k], p1s[k2], p2s[k2])
            pltpu.async_copy(xbs[k], out_dst(g), oss[k])

            @pl.when(g + 2 < NCH)
            def _():
                issue_pe(g + 2, k2)

            @pl.when(g - 2 >= 0)
            def _():
                pltpu.make_async_copy(xbs[kx2], out_dst(g - 2), oss[kx2]).wait()

            @pl.when(g + 2 < NCH)
            def _():
                pltpu.async_copy(x_src(g + 2), xbs[kx2], ixs[kx2])

        return 0

    lax.fori_loop(0, NIT, body, 0)

    pltpu.make_async_copy(xbs[2], out_dst(NCH - 2), oss[2]).wait()
    pltpu.make_async_copy(xbs[3], out_dst(NCH - 1), oss[3]).wait()


@jax.jit
def kernel(x, pos_x, pos_y, pe1, pe2):
    run = pl.kernel(
        _sc_body,
        out_type=jax.ShapeDtypeStruct((SEQ, BATCH, D), jnp.float32),
        mesh=plsc.VectorSubcoreMesh(core_axis_name="c", subcore_axis_name="s"),
        scratch_types=[
            pltpu.VMEM((RPW,), jnp.int32),
            pltpu.VMEM((RPW,), jnp.int32),
            pltpu.VMEM((CH, BATCH, D), jnp.float32),
            pltpu.VMEM((CH, BATCH, D), jnp.float32),
            pltpu.VMEM((CH, BATCH, D), jnp.float32),
            pltpu.VMEM((CH, BATCH, D), jnp.float32),
            pltpu.VMEM((CH, 1, D), jnp.float32),
            pltpu.VMEM((CH, 1, D), jnp.float32),
            pltpu.VMEM((CH, 1, D), jnp.float32),
            pltpu.VMEM((CH, 1, D), jnp.float32),
            pltpu.SemaphoreType.DMA,
            pltpu.SemaphoreType.DMA,
            pltpu.SemaphoreType.DMA,
            pltpu.SemaphoreType.DMA,
            pltpu.SemaphoreType.DMA,
            pltpu.SemaphoreType.DMA,
            pltpu.SemaphoreType.DMA,
            pltpu.SemaphoreType.DMA,
            pltpu.SemaphoreType.DMA,
            pltpu.SemaphoreType.DMA,
        ],
    )
    return run(x, pos_x, pos_y, pe1, pe2)


# speedup vs baseline: 29.0719x; 1.0932x over previous
import jax
import jax.numpy as jnp
from jax import lax
from jax.experimental import pallas as pl
from jax.experimental.pallas import tpu as pltpu
from jax.experimental.pallas import tpu_sc as plsc

SEQ = 8192
BATCH = 4
D = 768
HD = D // 2
L = 16

NC = 2
NS = 16
NW = NC * NS
RPW = SEQ // NW
CH = 8
NCH = RPW // CH
NIT = NCH // 4


def _sc_body(x_hbm, posx_hbm, posy_hbm, pe1_hbm, pe2_hbm, out_hbm,
             rawx_v, rawy_v,
             xb0, xb1, xb2, xb3, p10, p11, p20, p21,
             ix0, ix1, ix2, ix3, ip0, ip1, os0, os1, os2, os3):
    wid = lax.axis_index("s") * NC + lax.axis_index("c")
    base = wid * RPW

    xbs = (xb0, xb1, xb2, xb3)
    ixs = (ix0, ix1, ix2, ix3)
    oss = (os0, os1, os2, os3)
    p1s = (p10, p11)
    p2s = (p20, p21)
    ips = (ip0, ip1)

    pltpu.sync_copy(posx_hbm.at[pl.ds(base, RPW)], rawx_v)
    pltpu.sync_copy(posy_hbm.at[pl.ds(base, RPW)], rawy_v)

    def x_src(c):
        return x_hbm.at[pl.ds(base + c * CH, CH)]

    def out_dst(c):
        return out_hbm.at[pl.ds(base + c * CH, CH)]

    def pe_srcs(c):
        return (pe1_hbm.at[rawx_v.at[pl.ds(c * CH, CH)], pl.ds(0, 1), pl.ds(0, HD)],
                pe2_hbm.at[rawy_v.at[pl.ds(c * CH, CH)], pl.ds(0, 1), pl.ds(HD, HD)])

    def issue_pe(c, k2):
        s1, s2 = pe_srcs(c)
        pltpu.async_copy(s1, p1s[k2], ips[k2])
        pltpu.async_copy(s2, p2s[k2], ips[k2])

    def add_pe(xb, p1, p2):
        def row_add(r, _):
            for j in range(HD // L):
                v1 = p1[r, 0, pl.ds(j * L, L)]
                v2 = p2[r, 0, pl.ds(j * L, L)]
                for b in range(BATCH):
                    plsc.addupdate(xb.at[r, b, pl.ds(j * L, L)], v1)
                    plsc.addupdate(xb.at[r, b, pl.ds(HD + j * L, L)], v2)
            return 0

        lax.fori_loop(0, CH, row_add, 0)

    pltpu.async_copy(x_src(0), xb0, ix0)
    pltpu.async_copy(x_src(1), xb1, ix1)
    issue_pe(0, 0)
    issue_pe(1, 1)

    def body(it, _):
        g0 = 4 * it
        for k in range(4):
            g = g0 + k
            k2 = k % 2
            kx2 = (k + 2) % 4
            pltpu.make_async_copy(x_src(g), xbs[k], ixs[k]).wait()
            s1, s2 = pe_srcs(g)
            pltpu.make_async_copy(s1, p1s[k2], ips[k2]).wait()
            pltpu.make_async_copy(s2, p2s[k2], ips[k2]).wait()
            add_pe(xbs[k], p1s[k2], p2s[k2])
            pltpu.async_copy(xbs[k], out_dst(g), oss[k])

            @pl.when(g + 2 < NCH)
            def _():
                issue_pe(g + 2, k2)

            @pl.when(g - 2 >= 0)
            def _():
                pltpu.make_async_copy(xbs[kx2], out_dst(g - 2), oss[kx2]).wait()

            @pl.when(g + 2 < NCH)
            def _():
                pltpu.async_copy(x_src(g + 2), xbs[kx2], ixs[kx2])

        return 0

    lax.fori_loop(0, NIT, body, 0)

    pltpu.make_async_copy(xbs[2], out_dst(NCH - 2), oss[2]).wait()
    pltpu.make_async_copy(xbs[3], out_dst(NCH - 1), oss[3]).wait()


@jax.jit
def kernel(x, pos_x, pos_y, pe1, pe2):
    run = pl.kernel(
        _sc_body,
        out_type=jax.ShapeDtypeStruct((SEQ, BATCH, D), jnp.float32),
        mesh=plsc.VectorSubcoreMesh(core_axis_name="c", subcore_axis_name="s"),
        scratch_types=[
            pltpu.VMEM((RPW,), jnp.int32),
            pltpu.VMEM((RPW,), jnp.int32),
            pltpu.VMEM((CH, BATCH, D), jnp.float32),
            pltpu.VMEM((CH, BATCH, D), jnp.float32),
            pltpu.VMEM((CH, BATCH, D), jnp.float32),
            pltpu.VMEM((CH, BATCH, D), jnp.float32),
            pltpu.VMEM((CH, 1, HD), jnp.float32),
            pltpu.VMEM((CH, 1, HD), jnp.float32),
            pltpu.VMEM((CH, 1, HD), jnp.float32),
            pltpu.VMEM((CH, 1, HD), jnp.float32),
            pltpu.SemaphoreType.DMA,
            pltpu.SemaphoreType.DMA,
            pltpu.SemaphoreType.DMA,
            pltpu.SemaphoreType.DMA,
            pltpu.SemaphoreType.DMA,
            pltpu.SemaphoreType.DMA,
            pltpu.SemaphoreType.DMA,
            pltpu.SemaphoreType.DMA,
            pltpu.SemaphoreType.DMA,
            pltpu.SemaphoreType.DMA,
        ],
    )
    return run(x, pos_x, pos_y, pe1, pe2)


# speedup vs baseline: 30.0541x; 1.0338x over previous
import jax
import jax.numpy as jnp
from jax import lax
from jax.experimental import pallas as pl
from jax.experimental.pallas import tpu as pltpu
from jax.experimental.pallas import tpu_sc as plsc

SEQ = 8192
BATCH = 4
D = 768
HD = D // 2
L = 16

NC = 2
NS = 16
NW = NC * NS
RPW = SEQ // NW
CH = 4
NCH = RPW // CH
NIT = NCH // 4
PCH = 2 * CH


def _sc_body(x_hbm, posx_hbm, posy_hbm, pe1_hbm, pe2_hbm, out_hbm,
             rawx_v, rawy_v,
             xb0, xb1, xb2, xb3, p10, p11, p20, p21,
             sxb0, sxb1, sxb2, sxb3,
             ix0, ix1, ix2, ix3, ip0, ip1,
             bs0, bs1, bs2, bs3, os0, os1, os2, os3):
    cid = lax.axis_index("c")
    sid = lax.axis_index("s")
    wid = sid * NC + cid
    base = wid * RPW

    xbs = (xb0, xb1, xb2, xb3)
    sxbs = (sxb0, sxb1, sxb2, sxb3)
    ixs = (ix0, ix1, ix2, ix3)
    bss = (bs0, bs1, bs2, bs3)
    oss = (os0, os1, os2, os3)
    p1s = (p10, p11)
    p2s = (p20, p21)
    ips = (ip0, ip1)

    pltpu.sync_copy(posx_hbm.at[pl.ds(base, RPW)], rawx_v)
    pltpu.sync_copy(posy_hbm.at[pl.ds(base, RPW)], rawy_v)

    def x_src(c):
        return x_hbm.at[pl.ds(base + c * CH, CH)]

    def out_dst(c):
        return out_hbm.at[pl.ds(base + c * CH, CH)]

    def pe_srcs(P):
        return (pe1_hbm.at[rawx_v.at[pl.ds(P * PCH, PCH)], pl.ds(0, 1), pl.ds(0, HD)],
                pe2_hbm.at[rawy_v.at[pl.ds(P * PCH, PCH)], pl.ds(0, 1), pl.ds(HD, HD)])

    def issue_pe(P, slot):
        s1, s2 = pe_srcs(P)
        pltpu.async_copy(s1, p1s[slot], ips[slot])
        pltpu.async_copy(s2, p2s[slot], ips[slot])

    def wait_pe(P, slot):
        s1, s2 = pe_srcs(P)
        pltpu.make_async_copy(s1, p1s[slot], ips[slot]).wait()
        pltpu.make_async_copy(s2, p2s[slot], ips[slot]).wait()

    def add_pe(xb, p1, p2, off):
        def row_add(r, _):
            for j in range(HD // L):
                v1 = p1[off + r, 0, pl.ds(j * L, L)]
                v2 = p2[off + r, 0, pl.ds(j * L, L)]
                for b in range(BATCH):
                    plsc.addupdate(xb.at[r, b, pl.ds(j * L, L)], v1)
                    plsc.addupdate(xb.at[r, b, pl.ds(HD + j * L, L)], v2)
            return 0

        lax.fori_loop(0, CH, row_add, 0)

    pltpu.async_copy(x_src(0), xb0, ix0)
    pltpu.async_copy(x_src(1), xb1, ix1)
    pltpu.async_copy(x_src(2), xb2, ix2)
    issue_pe(0, 0)
    issue_pe(1, 1)

    def body(it, _):
        g0 = 4 * it
        for k in range(4):
            g = g0 + k
            slot = k // 2
            km1 = (k + 3) % 4
            P = g // 2

            pltpu.make_async_copy(x_src(g), xbs[k], ixs[k]).wait()
            if k % 2 == 0:
                wait_pe(P, slot)
            add_pe(xbs[k], p1s[slot], p2s[slot], (k % 2) * CH)

            @pl.when(g - 4 >= 0)
            def _():
                pltpu.make_async_copy(sxbs[k].at[sid], out_dst(g - 4), oss[k]).wait()

            pltpu.async_copy(xbs[k], sxbs[k].at[sid], bss[k])

            if k % 2 == 1:
                @pl.when(g <= NCH - 5)
                def _():
                    issue_pe(P + 2, slot)

            @pl.when(g - 1 >= 0)
            def _():
                pltpu.make_async_copy(xbs[km1], sxbs[km1].at[sid], bss[km1]).wait()
                pltpu.async_copy(sxbs[km1].at[sid], out_dst(g - 1), oss[km1])

            @pl.when(g + 3 < NCH)
            def _():
                pltpu.async_copy(x_src(g + 3), xbs[km1], ixs[km1])

        return 0

    lax.fori_loop(0, NIT, body, 0)

    pltpu.make_async_copy(xbs[3], sxbs[3].at[sid], bss[3]).wait()
    pltpu.async_copy(sxbs[3].at[sid], out_dst(NCH - 1), oss[3])
    for k in range(4):
        pltpu.make_async_copy(sxbs[k].at[sid], out_dst(NCH - 4 + k), oss[k]).wait()


@jax.jit
def kernel(x, pos_x, pos_y, pe1, pe2):
    run = pl.kernel(
        _sc_body,
        out_type=jax.ShapeDtypeStruct((SEQ, BATCH, D), jnp.float32),
        mesh=plsc.VectorSubcoreMesh(core_axis_name="c", subcore_axis_name="s"),
        scratch_types=[
            pltpu.VMEM((RPW,), jnp.int32),
            pltpu.VMEM((RPW,), jnp.int32),
            pltpu.VMEM((CH, BATCH, D), jnp.float32),
            pltpu.VMEM((CH, BATCH, D), jnp.float32),
            pltpu.VMEM((CH, BATCH, D), jnp.float32),
            pltpu.VMEM((CH, BATCH, D), jnp.float32),
            pltpu.VMEM((PCH, 1, HD), jnp.float32),
            pltpu.VMEM((PCH, 1, HD), jnp.float32),
            pltpu.VMEM((PCH, 1, HD), jnp.float32),
            pltpu.VMEM((PCH, 1, HD), jnp.float32),
            pltpu.VMEM_SHARED((NS, CH, BATCH, D), jnp.float32),
            pltpu.VMEM_SHARED((NS, CH, BATCH, D), jnp.float32),
            pltpu.VMEM_SHARED((NS, CH, BATCH, D), jnp.float32),
            pltpu.VMEM_SHARED((NS, CH, BATCH, D), jnp.float32),
            pltpu.SemaphoreType.DMA,
            pltpu.SemaphoreType.DMA,
            pltpu.SemaphoreType.DMA,
            pltpu.SemaphoreType.DMA,
            pltpu.SemaphoreType.DMA,
            pltpu.SemaphoreType.DMA,
            pltpu.SemaphoreType.DMA,
            pltpu.SemaphoreType.DMA,
            pltpu.SemaphoreType.DMA,
            pltpu.SemaphoreType.DMA,
            pltpu.SemaphoreType.DMA,
            pltpu.SemaphoreType.DMA,
            pltpu.SemaphoreType.DMA,
            pltpu.SemaphoreType.DMA,
        ],
    )
    return run(x, pos_x, pos_y, pe1, pe2)
